# Initial kernel scaffold; baseline (speedup 1.0000x reference)
import jax
import jax.numpy as jnp
from jax.experimental import pallas as pl


def kernel(pred, target):
    raise NotImplementedError("write your pallas kernel here")


# speedup vs baseline: 26.2582x; 26.2582x over previous
import functools

import jax
import jax.numpy as jnp
from jax import lax
from jax.experimental import pallas as pl
from jax.experimental.pallas import tpu as pltpu
from jax.experimental.pallas import tpu_sc as plsc

N = 32 * 512 * 512
B = 1024
L = 16
NC = 2
NS = 16
NW = NC * NS
PER_W = N // NW
CHUNK = 16384
NCHUNK = PER_W // CHUNK


def _sc_hist(pred_hbm, tgt_hbm, out_hbm, pbuf, tbuf, cnt, pos, esum, sem):
    cid = lax.axis_index("c")
    sid = lax.axis_index("s")
    wid = sid * NC + cid
    base = wid * PER_W

    zeros = jnp.zeros((L,), jnp.float32)
    ones = jnp.ones((L,), jnp.float32)
    lane_base = lax.iota(jnp.int32, L) * B

    def zero_body(k, _):
        cnt[pl.ds(k * L, L)] = zeros
        pos[pl.ds(k * L, L)] = zeros
        esum[pl.ds(k * L, L)] = zeros
        return 0

    lax.fori_loop(0, (L * B) // L, zero_body, 0)

    fB = jnp.float32(B)

    def chunk_body(ci, _):
        off = base + ci * CHUNK
        pltpu.sync_copy(pred_hbm.at[pl.ds(off, CHUNK)], pbuf)
        pltpu.sync_copy(tgt_hbm.at[pl.ds(off, CHUNK)], tbuf)

        def vec_body(i, _):
            x = pbuf[pl.ds(i * L, L)]
            t = tbuf[pl.ds(i * L, L)]
            s = x * (t + t - 1.0)
            e = 1.0 / (1.0 + jnp.exp(s))
            b = jnp.minimum((e * fB).astype(jnp.int32), B - 1)
            addr = lane_base + b
            plsc.addupdate_scatter(cnt, [addr], ones)
            plsc.addupdate_scatter(pos, [addr], t)
            plsc.addupdate_scatter(esum, [addr], e)
            return 0

        lax.fori_loop(0, CHUNK // L, vec_body, 0)
        return 0

    lax.fori_loop(0, NCHUNK, chunk_body, 0)

    pltpu.sync_copy(cnt, out_hbm.at[0, wid])
    pltpu.sync_copy(pos, out_hbm.at[1, wid])
    pltpu.sync_copy(esum, out_hbm.at[2, wid])


_sc_hist_call = functools.partial(
    pl.kernel,
    out_type=jax.ShapeDtypeStruct((3, NW, L * B), jnp.float32),
    mesh=plsc.VectorSubcoreMesh(core_axis_name="c", subcore_axis_name="s"),
    scratch_types=[
        pltpu.VMEM((CHUNK,), jnp.float32),
        pltpu.VMEM((CHUNK,), jnp.float32),
        pltpu.VMEM((L * B,), jnp.float32),
        pltpu.VMEM((L * B,), jnp.float32),
        pltpu.VMEM((L * B,), jnp.float32),
        pltpu.SemaphoreType.DMA,
    ],
    compiler_params=pltpu.CompilerParams(needs_layout_passes=False),
)(_sc_hist)


def _tc_finish(h_ref, o_ref):
    H = h_ref[...]
    S = jnp.sum(H, axis=1)
    S2 = S[:, 0:B]
    for l in range(1, L):
        S2 = S2 + S[:, l * B:(l + 1) * B]
    cnt = S2[0:1]
    pos = S2[1:2]
    esum = S2[2:3]

    r = lax.broadcasted_iota(jnp.int32, (B, B), 0)
    c = lax.broadcasted_iota(jnp.int32, (B, B), 1)
    mlow = (r >= c).astype(jnp.float32)

    suf_cnt = jnp.dot(cnt, mlow, precision=lax.Precision.HIGHEST)
    suf_pos = jnp.dot(pos, mlow, precision=lax.Precision.HIGHEST)
    p_tot = jnp.sum(pos, axis=1, keepdims=True)

    def jac(i_, c1):
        den = p_tot + i_ - c1
        den = jnp.where(den == 0.0, 1.0, den)
        return jnp.where(i_ == 0.0, 0.0, 1.0 - (p_tot - c1) / den)

    j_end = jac(suf_cnt, suf_pos)
    j_start = jac(suf_cnt - cnt, suf_pos - pos)
    contrib = esum / jnp.maximum(cnt, 1.0) * (j_end - j_start)
    o_ref[...] = jnp.sum(contrib, axis=1, keepdims=True)


_tc_finish_call = pl.pallas_call(
    _tc_finish,
    out_shape=jax.ShapeDtypeStruct((1, 1), jnp.float32),
)


def kernel(pred, target):
    p = pred.reshape(N)
    t = target.reshape(N)
    hist = _sc_hist_call(p, t)
    loss = _tc_finish_call(hist)
    return loss.reshape(())


# speedup vs baseline: 27.6200x; 1.0519x over previous
import functools

import jax
import jax.numpy as jnp
from jax import lax
from jax.experimental import pallas as pl
from jax.experimental.pallas import tpu as pltpu
from jax.experimental.pallas import tpu_sc as plsc

N = 32 * 512 * 512
B = 1024
L = 16
NC = 2
NS = 16
NW = NC * NS
PER_W = N // NW
CHUNK = 16384
NCHUNK = PER_W // CHUNK


def _sc_hist(pred_hbm, tgt_hbm, out_hbm,
             pbuf0, tbuf0, pbuf1, tbuf1, cnt, pos, esum, semA, semB):
    cid = lax.axis_index("c")
    sid = lax.axis_index("s")
    wid = sid * NC + cid
    base = wid * PER_W

    zeros = jnp.zeros((L,), jnp.float32)
    ones = jnp.ones((L,), jnp.float32)
    lane_base = lax.iota(jnp.int32, L) * B
    fB = jnp.float32(B)

    def zero_body(k, _):
        cnt[pl.ds(k * L, L)] = zeros
        pos[pl.ds(k * L, L)] = zeros
        esum[pl.ds(k * L, L)] = zeros
        return 0

    lax.fori_loop(0, (L * B) // L, zero_body, 0)

    def start_fetch(ci, pbuf, tbuf, sem):
        off = base + ci * CHUNK
        pltpu.async_copy(pred_hbm.at[pl.ds(off, CHUNK)], pbuf, sem)
        pltpu.async_copy(tgt_hbm.at[pl.ds(off, CHUNK)], tbuf, sem)

    def wait_fetch(pbuf, tbuf, sem):
        pltpu.make_async_copy(pred_hbm.at[pl.ds(0, CHUNK)], pbuf, sem).wait()
        pltpu.make_async_copy(tgt_hbm.at[pl.ds(0, CHUNK)], tbuf, sem).wait()

    def process(pbuf, tbuf):
        def vec_body(i, _):
            x = pbuf[pl.ds(i * L, L)]
            t = tbuf[pl.ds(i * L, L)]
            s = x * (t + t - 1.0)
            e = 1.0 / (1.0 + jnp.exp(s))
            b = jnp.minimum((e * fB).astype(jnp.int32), B - 1)
            addr = lane_base + b
            plsc.addupdate_scatter(cnt, [addr], ones)
            plsc.addupdate_scatter(pos, [addr], t)
            plsc.addupdate_scatter(esum, [addr], e)
            return 0

        lax.fori_loop(0, CHUNK // L, vec_body, 0, unroll=8)

    start_fetch(0, pbuf0, tbuf0, semA)

    def pair_body(g, _):
        ci0 = g * 2
        start_fetch(ci0 + 1, pbuf1, tbuf1, semB)
        wait_fetch(pbuf0, tbuf0, semA)
        process(pbuf0, tbuf0)

        @pl.when(ci0 + 2 < NCHUNK)
        def _():
            start_fetch(ci0 + 2, pbuf0, tbuf0, semA)

        wait_fetch(pbuf1, tbuf1, semB)
        process(pbuf1, tbuf1)
        return 0

    lax.fori_loop(0, NCHUNK // 2, pair_body, 0)

    pltpu.sync_copy(cnt, out_hbm.at[0, wid])
    pltpu.sync_copy(pos, out_hbm.at[1, wid])
    pltpu.sync_copy(esum, out_hbm.at[2, wid])


_sc_hist_call = functools.partial(
    pl.kernel,
    out_type=jax.ShapeDtypeStruct((3, NW, L * B), jnp.float32),
    mesh=plsc.VectorSubcoreMesh(core_axis_name="c", subcore_axis_name="s"),
    scratch_types=[
        pltpu.VMEM((CHUNK,), jnp.float32),
        pltpu.VMEM((CHUNK,), jnp.float32),
        pltpu.VMEM((CHUNK,), jnp.float32),
        pltpu.VMEM((CHUNK,), jnp.float32),
        pltpu.VMEM((L * B,), jnp.float32),
        pltpu.VMEM((L * B,), jnp.float32),
        pltpu.VMEM((L * B,), jnp.float32),
        pltpu.SemaphoreType.DMA,
        pltpu.SemaphoreType.DMA,
    ],
    compiler_params=pltpu.CompilerParams(needs_layout_passes=False),
)(_sc_hist)


def _tc_finish(h_ref, o_ref):
    H = h_ref[...]
    S = jnp.sum(H, axis=1)
    S2 = S[:, 0:B]
    for l in range(1, L):
        S2 = S2 + S[:, l * B:(l + 1) * B]
    cnt = S2[0:1]
    pos = S2[1:2]
    esum = S2[2:3]

    r = lax.broadcasted_iota(jnp.int32, (B, B), 0)
    c = lax.broadcasted_iota(jnp.int32, (B, B), 1)
    mlow = (r >= c).astype(jnp.float32)

    suf_cnt = jnp.dot(cnt, mlow, precision=lax.Precision.HIGHEST)
    suf_pos = jnp.dot(pos, mlow, precision=lax.Precision.HIGHEST)
    p_tot = jnp.sum(pos, axis=1, keepdims=True)

    def jac(i_, c1):
        den = p_tot + i_ - c1
        den = jnp.where(den == 0.0, 1.0, den)
        return jnp.where(i_ == 0.0, 0.0, 1.0 - (p_tot - c1) / den)

    j_end = jac(suf_cnt, suf_pos)
    j_start = jac(suf_cnt - cnt, suf_pos - pos)
    contrib = esum / jnp.maximum(cnt, 1.0) * (j_end - j_start)
    o_ref[...] = jnp.sum(contrib, axis=1, keepdims=True)


_tc_finish_call = pl.pallas_call(
    _tc_finish,
    out_shape=jax.ShapeDtypeStruct((1, 1), jnp.float32),
)


def kernel(pred, target):
    p = pred.reshape(N)
    t = target.reshape(N)
    hist = _sc_hist_call(p, t)
    loss = _tc_finish_call(hist)
    return loss.reshape(())


# speedup vs baseline: 77.8187x; 2.8175x over previous
import functools

import jax
import jax.numpy as jnp
from jax import lax
from jax.experimental import pallas as pl
from jax.experimental.pallas import tpu as pltpu
from jax.experimental.pallas import tpu_sc as plsc

N = 32 * 512 * 512
B = 1024
L = 16
NC = 2
NS = 16
NW = NC * NS
PER_W = N // NW
CHUNK = 16384
NCHUNK = PER_W // CHUNK


def _sc_hist(pred_hbm, tgt_hbm, out_hbm,
             pbuf0, tbuf0, pbuf1, tbuf1, cnt, pos, esum, semA, semB):
    cid = lax.axis_index("c")
    sid = lax.axis_index("s")
    wid = sid * NC + cid
    base = wid * PER_W

    zeros = jnp.zeros((L,), jnp.float32)
    ones = jnp.ones((L,), jnp.float32)
    lane_base = lax.iota(jnp.int32, L) * B
    fB = jnp.float32(B)

    def zero_body(k, _):
        cnt[pl.ds(k * L, L)] = zeros
        pos[pl.ds(k * L, L)] = zeros
        esum[pl.ds(k * L, L)] = zeros
        return 0

    lax.fori_loop(0, (L * B) // L, zero_body, 0)

    def start_fetch(ci, pbuf, tbuf, sem):
        off = base + ci * CHUNK
        pltpu.async_copy(pred_hbm.at[pl.ds(off, CHUNK)], pbuf, sem)
        pltpu.async_copy(tgt_hbm.at[pl.ds(off, CHUNK)], tbuf, sem)

    def wait_fetch(pbuf, tbuf, sem):
        pltpu.make_async_copy(pred_hbm.at[pl.ds(0, CHUNK)], pbuf, sem).wait()
        pltpu.make_async_copy(tgt_hbm.at[pl.ds(0, CHUNK)], tbuf, sem).wait()

    U = 8
    LOG2E = 1.4426950408889634

    def process(pbuf, tbuf):
        def vec_body(i, _):
            xs = [pbuf[pl.ds((i * U + u) * L, L)] for u in range(U)]
            ts = [tbuf[pl.ds((i * U + u) * L, L)] for u in range(U)]
            ss = [x * (t + t - 1.0) for x, t in zip(xs, ts)]
            exs = [jnp.exp(s) for s in ss]
            es = [1.0 / (1.0 + ex) for ex in exs]
            bs = [jnp.minimum((e * fB).astype(jnp.int32), B - 1) for e in es]
            addrs = [lane_base + b for b in bs]
            for u in range(U):
                plsc.addupdate_scatter(cnt, [addrs[u]], ones)
                plsc.addupdate_scatter(pos, [addrs[u]], ts[u])
                plsc.addupdate_scatter(esum, [addrs[u]], es[u])
            return 0

        lax.fori_loop(0, CHUNK // (L * U), vec_body, 0)

    start_fetch(0, pbuf0, tbuf0, semA)

    def pair_body(g, _):
        ci0 = g * 2
        start_fetch(ci0 + 1, pbuf1, tbuf1, semB)
        wait_fetch(pbuf0, tbuf0, semA)
        process(pbuf0, tbuf0)

        @pl.when(ci0 + 2 < NCHUNK)
        def _():
            start_fetch(ci0 + 2, pbuf0, tbuf0, semA)

        wait_fetch(pbuf1, tbuf1, semB)
        process(pbuf1, tbuf1)
        return 0

    lax.fori_loop(0, NCHUNK // 2, pair_body, 0)

    pltpu.sync_copy(cnt, out_hbm.at[0, wid])
    pltpu.sync_copy(pos, out_hbm.at[1, wid])
    pltpu.sync_copy(esum, out_hbm.at[2, wid])


_sc_hist_call = functools.partial(
    pl.kernel,
    out_type=jax.ShapeDtypeStruct((3, NW, L * B), jnp.float32),
    mesh=plsc.VectorSubcoreMesh(core_axis_name="c", subcore_axis_name="s"),
    scratch_types=[
        pltpu.VMEM((CHUNK,), jnp.float32),
        pltpu.VMEM((CHUNK,), jnp.float32),
        pltpu.VMEM((CHUNK,), jnp.float32),
        pltpu.VMEM((CHUNK,), jnp.float32),
        pltpu.VMEM((L * B,), jnp.float32),
        pltpu.VMEM((L * B,), jnp.float32),
        pltpu.VMEM((L * B,), jnp.float32),
        pltpu.SemaphoreType.DMA,
        pltpu.SemaphoreType.DMA,
    ],
    compiler_params=pltpu.CompilerParams(needs_layout_passes=False),
)(_sc_hist)


def _tc_finish(h_ref, o_ref):
    H = h_ref[...]
    S = jnp.sum(H, axis=1)
    S2 = S[:, 0:B]
    for l in range(1, L):
        S2 = S2 + S[:, l * B:(l + 1) * B]
    cnt = S2[0:1]
    pos = S2[1:2]
    esum = S2[2:3]

    r = lax.broadcasted_iota(jnp.int32, (B, B), 0)
    c = lax.broadcasted_iota(jnp.int32, (B, B), 1)
    mlow = (r >= c).astype(jnp.float32)

    suf_cnt = jnp.dot(cnt, mlow, precision=lax.Precision.HIGHEST)
    suf_pos = jnp.dot(pos, mlow, precision=lax.Precision.HIGHEST)
    p_tot = jnp.sum(pos, axis=1, keepdims=True)

    def jac(i_, c1):
        den = p_tot + i_ - c1
        den = jnp.where(den == 0.0, 1.0, den)
        return jnp.where(i_ == 0.0, 0.0, 1.0 - (p_tot - c1) / den)

    j_end = jac(suf_cnt, suf_pos)
    j_start = jac(suf_cnt - cnt, suf_pos - pos)
    contrib = esum / jnp.maximum(cnt, 1.0) * (j_end - j_start)
    o_ref[...] = jnp.sum(contrib, axis=1, keepdims=True)


_tc_finish_call = pl.pallas_call(
    _tc_finish,
    out_shape=jax.ShapeDtypeStruct((1, 1), jnp.float32),
)


def kernel(pred, target):
    p = pred.reshape(N)
    t = target.reshape(N)
    hist = _sc_hist_call(p, t)
    loss = _tc_finish_call(hist)
    return loss.reshape(())


# speedup vs baseline: 178.2577x; 2.2907x over previous
import functools

import jax
import jax.numpy as jnp
from jax import lax
from jax.experimental import pallas as pl
from jax.experimental.pallas import tpu as pltpu
from jax.experimental.pallas import tpu_sc as plsc

N = 32 * 512 * 512
B = 1024
L = 16
NC = 2
NS = 16
NW = NC * NS
PER_W = N // NW
CHUNK = 16384
NCHUNK = PER_W // CHUNK


ROWS = CHUNK // 512


def _sc_hist(pred_hbm, tgt_hbm, outc_hbm, oute_hbm,
             pbuf0, tbuf0, pbuf1, tbuf1, cntpos, esum, semA, semB):
    cid = lax.axis_index("c")
    sid = lax.axis_index("s")
    wid = sid * NC + cid

    zeros = jnp.zeros((L,), jnp.float32)
    izeros = jnp.zeros((L,), jnp.int32)
    lane_base = lax.iota(jnp.int32, L) * (B + 1) - 0x4B000000
    fB = jnp.float32(B)
    magic = jnp.float32(8388608.0 - 0.5)

    def zero_body(k, _):
        cntpos[pl.ds(k * L, L)] = izeros
        esum[pl.ds(k * L, L)] = zeros
        return 0

    lax.fori_loop(0, (L * (B + 1)) // L, zero_body, 0)

    def start_fetch(ci, pbuf, tbuf, sem):
        r0 = ci * ROWS
        pltpu.async_copy(pred_hbm.at[wid, pl.ds(r0, ROWS), :], pbuf, sem)
        pltpu.async_copy(tgt_hbm.at[wid, pl.ds(r0, ROWS), :], tbuf, sem)

    def wait_fetch(pbuf, tbuf, sem):
        pltpu.make_async_copy(pred_hbm.at[0, pl.ds(0, ROWS), :], pbuf, sem).wait()
        pltpu.make_async_copy(tgt_hbm.at[0, pl.ds(0, ROWS), :], tbuf, sem).wait()

    U = 16

    def process(pbuf, tbuf):
        def vec_body(i, _):
            row = i >> 1
            cb = (i & 1) * 256
            xs = [pbuf[row, pl.ds(cb + u * L, L)] for u in range(U)]
            ts = [tbuf[row, pl.ds(cb + u * L, L)] for u in range(U)]
            sgns = [lax.shift_left(plsc.bitcast(t, jnp.int32), 8) for t in ts]
            ss = [plsc.bitcast(lax.bitwise_xor(plsc.bitcast(x, jnp.int32), g),
                               jnp.float32) for x, g in zip(xs, sgns)]
            exs = [jnp.exp(s) for s in ss]
            es = [1.0 / (1.0 + ex) for ex in exs]
            addrs = [lane_base + plsc.bitcast(e * fB + magic, jnp.int32)
                     for e in es]
            cpvals = [lax.shift_right_logical(g, 16) + 1 for g in sgns]
            for u in range(U):
                plsc.addupdate_scatter(cntpos, [addrs[u]], cpvals[u])
                plsc.addupdate_scatter(esum, [addrs[u]], es[u])
            return 0

        lax.fori_loop(0, CHUNK // (L * U), vec_body, 0)

    start_fetch(0, pbuf0, tbuf0, semA)

    def pair_body(g, _):
        ci0 = g * 2
        start_fetch(ci0 + 1, pbuf1, tbuf1, semB)
        wait_fetch(pbuf0, tbuf0, semA)
        process(pbuf0, tbuf0)

        @pl.when(ci0 + 2 < NCHUNK)
        def _():
            start_fetch(ci0 + 2, pbuf0, tbuf0, semA)

        wait_fetch(pbuf1, tbuf1, semB)
        process(pbuf1, tbuf1)
        return 0

    lax.fori_loop(0, NCHUNK // 2, pair_body, 0)

    pltpu.sync_copy(cntpos, outc_hbm.at[wid])
    pltpu.sync_copy(esum, oute_hbm.at[wid])


B1 = B + 1

_sc_hist_call = functools.partial(
    pl.kernel,
    out_type=(jax.ShapeDtypeStruct((NW, L * B1), jnp.int32),
              jax.ShapeDtypeStruct((NW, L * B1), jnp.float32)),
    mesh=plsc.VectorSubcoreMesh(core_axis_name="c", subcore_axis_name="s"),
    scratch_types=[
        pltpu.VMEM((ROWS, 512), jnp.float32),
        pltpu.VMEM((ROWS, 512), jnp.float32),
        pltpu.VMEM((ROWS, 512), jnp.float32),
        pltpu.VMEM((ROWS, 512), jnp.float32),
        pltpu.VMEM((L * B1,), jnp.int32),
        pltpu.VMEM((L * B1,), jnp.float32),
        pltpu.SemaphoreType.DMA,
        pltpu.SemaphoreType.DMA,
    ],
    compiler_params=pltpu.CompilerParams(needs_layout_passes=False),
)(_sc_hist)


def _tc_finish(hc_ref, he_ref, o_ref):
    M = hc_ref[...]
    cnt_i = lax.bitwise_and(M, 32767)
    pos_i = lax.shift_right_logical(M, 15)
    cnt_f = jnp.sum(cnt_i, axis=0, keepdims=True).astype(jnp.float32)
    pos_f = jnp.sum(pos_i, axis=0, keepdims=True).astype(jnp.float32)
    ec_f = jnp.sum(he_ref[...], axis=0, keepdims=True)

    def lane_reduce(v):
        acc = v[:, 0:B1]
        for l in range(1, L):
            acc = acc + v[:, l * B1:(l + 1) * B1]
        return acc

    cnt = lane_reduce(cnt_f)
    pos = lane_reduce(pos_f)
    ecsum = lane_reduce(ec_f)
    esum = cnt - ecsum

    r = lax.broadcasted_iota(jnp.int32, (B1, B1), 0)
    c = lax.broadcasted_iota(jnp.int32, (B1, B1), 1)
    mup = (r <= c).astype(jnp.float32)

    pre_cnt = jnp.dot(cnt, mup, precision=lax.Precision.HIGHEST)
    pre_pos = jnp.dot(pos, mup, precision=lax.Precision.HIGHEST)
    p_tot = jnp.sum(pos, axis=1, keepdims=True)

    def jac(i_, c1):
        den = p_tot + i_ - c1
        den = jnp.where(den == 0.0, 1.0, den)
        return jnp.where(i_ == 0.0, 0.0, 1.0 - (p_tot - c1) / den)

    j_end = jac(pre_cnt, pre_pos)
    j_start = jac(pre_cnt - cnt, pre_pos - pos)
    contrib = esum / jnp.maximum(cnt, 1.0) * (j_end - j_start)
    o_ref[...] = jnp.sum(contrib, axis=1, keepdims=True)


_tc_finish_call = pl.pallas_call(
    _tc_finish,
    out_shape=jax.ShapeDtypeStruct((1, 1), jnp.float32),
)


def kernel(pred, target):
    hist_cp, hist_e = _sc_hist_call(pred, target)
    loss = _tc_finish_call(hist_cp, hist_e)
    return loss.reshape(())
